# baseline (speedup 1.0000x reference)
import functools

import jax
import jax.numpy as jnp
from jax import lax
from jax.experimental import pallas as pl
from jax.experimental.pallas import tpu as pltpu
from jax.experimental.pallas import tpu_sc as plsc

BATCH = 16384
EDIM = 64
NW = 32
BPW = BATCH // NW
CHUNK = 128
NCHUNK = BPW // CHUNK


def _gather_body(user_table, item_table, uidx, iidx, out_u, out_i,
                 uidx_v, iidx_v, urows_v, irows_v, sem):
    wid = lax.axis_index("s") * 2 + lax.axis_index("c")
    base = wid * BPW
    pltpu.sync_copy(uidx.at[wid], uidx_v)
    pltpu.sync_copy(iidx.at[wid], iidx_v)
    copies = []
    for c in range(NCHUNK):
        copies.append(pltpu.async_copy(
            user_table.at[uidx_v.at[c]],
            urows_v.at[pl.ds(c * CHUNK, CHUNK)], sem))
        copies.append(pltpu.async_copy(
            item_table.at[iidx_v.at[c]],
            irows_v.at[pl.ds(c * CHUNK, CHUNK)], sem))
    for cp in copies:
        cp.wait()
    pltpu.sync_copy(urows_v, out_u.at[pl.ds(base, BPW)])
    pltpu.sync_copy(irows_v, out_i.at[pl.ds(base, BPW)])


def _sc_gather(user_table, item_table, uidx, iidx):
    mesh = plsc.VectorSubcoreMesh(core_axis_name="c", subcore_axis_name="s")
    return pl.kernel(
        _gather_body,
        out_type=(
            jax.ShapeDtypeStruct((BATCH, EDIM), jnp.float32),
            jax.ShapeDtypeStruct((BATCH, EDIM), jnp.float32),
        ),
        mesh=mesh,
        scratch_types=[
            pltpu.VMEM((NCHUNK, CHUNK), jnp.int32),
            pltpu.VMEM((NCHUNK, CHUNK), jnp.int32),
            pltpu.VMEM((BPW, EDIM), jnp.float32),
            pltpu.VMEM((BPW, EDIM), jnp.float32),
            pltpu.SemaphoreType.DMA,
        ],
        compiler_params=pltpu.CompilerParams(use_tc_tiling_on_sc=False),
    )(user_table, item_table, uidx, iidx)


def _mlp_body(ue_ref, ie_ref, w1_ref, b1_ref, w2_ref, b2_ref,
              w3_ref, b3_ref, w4_ref, b4_ref, out_ref):
    dn = (((1,), (1,)), ((), ()))
    ue = ue_ref[...]
    ie = ie_ref[...]
    w1 = w1_ref[...]
    h = (lax.dot_general(ue, w1[:, :EDIM], dn,
                         preferred_element_type=jnp.float32)
         + lax.dot_general(ie, w1[:, EDIM:], dn,
                           preferred_element_type=jnp.float32)
         + b1_ref[...])
    h = jnp.maximum(h, 0.0)
    h = lax.dot_general(h, w2_ref[...], dn,
                        preferred_element_type=jnp.float32) + b2_ref[...]
    h = jnp.maximum(h, 0.0)
    h = lax.dot_general(h, w3_ref[...], dn,
                        preferred_element_type=jnp.float32) + b3_ref[...]
    h = jnp.maximum(h, 0.0)
    out_ref[...] = (jnp.sum(h * w4_ref[...], axis=1, keepdims=True)
                    + b4_ref[0, 0])


def _tc_mlp(ue, ie, W1, b1, W2, b2, W3, b3, W4, b4, block=2048):
    nblk = BATCH // block
    full = lambda shape: pl.BlockSpec(shape, lambda i: (0,) * len(shape))
    return pl.pallas_call(
        _mlp_body,
        grid=(nblk,),
        in_specs=[
            pl.BlockSpec((block, EDIM), lambda i: (i, 0)),
            pl.BlockSpec((block, EDIM), lambda i: (i, 0)),
            full(W1.shape), full(b1.shape),
            full(W2.shape), full(b2.shape),
            full(W3.shape), full(b3.shape),
            full(W4.shape), full(b4.shape),
        ],
        out_specs=pl.BlockSpec((block, 1), lambda i: (i, 0)),
        out_shape=jax.ShapeDtypeStruct((BATCH, 1), jnp.float32),
        compiler_params=pltpu.CompilerParams(
            dimension_semantics=("arbitrary",),
        ),
    )(ue, ie, W1, b1, W2, b2, W3, b3, W4, b4)


@jax.jit
def kernel(user_idx, item_idx, user_table, item_table,
           W1, b1, W2, b2, W3, b3, W4, b4):
    uidx = user_idx.astype(jnp.int32).reshape(NW, NCHUNK, CHUNK)
    iidx = item_idx.astype(jnp.int32).reshape(NW, NCHUNK, CHUNK)
    ue, ie = _sc_gather(user_table, item_table, uidx, iidx)
    b1r = b1.reshape(1, -1)
    b2r = b2.reshape(1, -1)
    b3r = b3.reshape(1, -1)
    b4r = b4.reshape(1, -1)
    return _tc_mlp(ue, ie, W1, b1r, W2, b2r, W3, b3r, W4, b4r)


# speedup vs baseline: 1.2021x; 1.2021x over previous
import jax
import jax.numpy as jnp
from jax import lax
from jax.experimental import pallas as pl
from jax.experimental.pallas import tpu as pltpu
from jax.experimental.pallas import tpu_sc as plsc

BATCH = 16384
EDIM = 64
NROWS = 1000000
PSPLIT = 500224
PBLK = 512
NBLK = PSPLIT // PBLK
NW = 32
BPW = BATCH // NW
CHUNK = 128
NCHUNK = BPW // CHUNK


def _relayout_body(u1, u2, i1, i2, pu, pi):
    pu[...] = jnp.concatenate([u1[...].T, u2[...].T], axis=1)
    pi[...] = jnp.concatenate([i1[...].T, i2[...].T], axis=1)


def _pair_tables(uT, iT):
    spec_lo = pl.BlockSpec((EDIM, PBLK), lambda i: (0, i))
    spec_hi = pl.BlockSpec((EDIM, PBLK), lambda i: (0, i + NBLK))
    out_spec = pl.BlockSpec((PBLK, 128), lambda i: (i, 0))
    return pl.pallas_call(
        _relayout_body,
        grid=(NBLK,),
        in_specs=[spec_lo, spec_hi, spec_lo, spec_hi],
        out_specs=[out_spec, out_spec],
        out_shape=[jax.ShapeDtypeStruct((PSPLIT, 128), jnp.float32)] * 2,
        compiler_params=pltpu.CompilerParams(
            dimension_semantics=("arbitrary",),
        ),
    )(uT, uT, iT, iT)


def _gather_body(ptab_u, ptab_i, uidx, iidx, out_u, out_i,
                 idx_v, rows_v, sem):
    wid = lax.axis_index("s") * 2 + lax.axis_index("c")
    base = wid * BPW

    def one_table(idx_hbm, ptab, out):
        pltpu.sync_copy(idx_hbm.at[wid], idx_v)
        for c in range(NCHUNK):
            for v in range(CHUNK // 16):
                sl = pl.ds(v * 16, 16)
                x = idx_v[c, sl]
                q = jnp.where(x >= PSPLIT, x - PSPLIT, x)
                idx_v[c, sl] = q
        cps = []
        for c in range(NCHUNK):
            cps.append(pltpu.async_copy(
                ptab.at[idx_v.at[c]],
                rows_v.at[pl.ds(c * CHUNK, CHUNK)], sem))
        for cp in cps:
            cp.wait()
        pltpu.sync_copy(rows_v, out.at[pl.ds(base, BPW)])

    one_table(uidx, ptab_u, out_u)
    one_table(iidx, ptab_i, out_i)


def _sc_gather(ptab_u, ptab_i, uidx, iidx):
    mesh = plsc.VectorSubcoreMesh(core_axis_name="c", subcore_axis_name="s")
    return pl.kernel(
        _gather_body,
        out_type=(
            jax.ShapeDtypeStruct((BATCH, 128), jnp.float32),
            jax.ShapeDtypeStruct((BATCH, 128), jnp.float32),
        ),
        mesh=mesh,
        scratch_types=[
            pltpu.VMEM((NCHUNK, CHUNK), jnp.int32),
            pltpu.VMEM((BPW, 128), jnp.float32),
            pltpu.SemaphoreType.DMA,
        ],
    )(ptab_u, ptab_i, uidx, iidx)


def _mlp_body(ur_ref, ir_ref, uh_ref, ih_ref, w1_ref, b1_ref, w2_ref, b2_ref,
              w3_ref, b3_ref, w4_ref, b4_ref, out_ref):
    dn = (((1,), (1,)), ((), ()))
    ur = ur_ref[...]
    ir = ir_ref[...]
    ue = jnp.where(uh_ref[...] >= PSPLIT, ur[:, EDIM:], ur[:, :EDIM])
    ie = jnp.where(ih_ref[...] >= PSPLIT, ir[:, EDIM:], ir[:, :EDIM])
    w1 = w1_ref[...]
    h = (lax.dot_general(ue, w1[:, :EDIM], dn,
                         preferred_element_type=jnp.float32)
         + lax.dot_general(ie, w1[:, EDIM:], dn,
                           preferred_element_type=jnp.float32)
         + b1_ref[...])
    h = jnp.maximum(h, 0.0)
    h = lax.dot_general(h, w2_ref[...], dn,
                        preferred_element_type=jnp.float32) + b2_ref[...]
    h = jnp.maximum(h, 0.0)
    h = lax.dot_general(h, w3_ref[...], dn,
                        preferred_element_type=jnp.float32) + b3_ref[...]
    h = jnp.maximum(h, 0.0)
    out_ref[...] = (jnp.sum(h * w4_ref[...], axis=1, keepdims=True)
                    + b4_ref[0, 0])


def _tc_mlp(ur, ir, uidx2, iidx2, W1, b1, W2, b2, W3, b3, W4, b4, block=2048):
    nblk = BATCH // block
    full = lambda shape: pl.BlockSpec(shape, lambda i: (0,) * len(shape))
    return pl.pallas_call(
        _mlp_body,
        grid=(nblk,),
        in_specs=[
            pl.BlockSpec((block, 128), lambda i: (i, 0)),
            pl.BlockSpec((block, 128), lambda i: (i, 0)),
            pl.BlockSpec((block, 1), lambda i: (i, 0)),
            pl.BlockSpec((block, 1), lambda i: (i, 0)),
            full(W1.shape), full(b1.shape),
            full(W2.shape), full(b2.shape),
            full(W3.shape), full(b3.shape),
            full(W4.shape), full(b4.shape),
        ],
        out_specs=pl.BlockSpec((block, 1), lambda i: (i, 0)),
        out_shape=jax.ShapeDtypeStruct((BATCH, 1), jnp.float32),
        compiler_params=pltpu.CompilerParams(
            dimension_semantics=("arbitrary",),
        ),
    )(ur, ir, uidx2, iidx2, W1, b1, W2, b2, W3, b3, W4, b4)


@jax.jit
def kernel(user_idx, item_idx, user_table, item_table,
           W1, b1, W2, b2, W3, b3, W4, b4):
    uT = user_table.T
    iT = item_table.T
    ptab_u, ptab_i = _pair_tables(uT, iT)
    uidx = user_idx.astype(jnp.int32)
    iidx = item_idx.astype(jnp.int32)
    ur, ir = _sc_gather(ptab_u, ptab_i,
                        uidx.reshape(NW, NCHUNK, CHUNK),
                        iidx.reshape(NW, NCHUNK, CHUNK))
    return _tc_mlp(ur, ir,
                   uidx.reshape(BATCH, 1), iidx.reshape(BATCH, 1),
                   W1, b1.reshape(1, -1), W2, b2.reshape(1, -1),
                   W3, b3.reshape(1, -1), W4, b4.reshape(1, -1))


# speedup vs baseline: 1.4012x; 1.1656x over previous
import jax
import jax.numpy as jnp
from jax import lax
from jax.experimental import pallas as pl
from jax.experimental.pallas import tpu as pltpu
from jax.experimental.pallas import tpu_sc as plsc

BATCH = 16384
EDIM = 64
NROWS = 1000000
PSPLIT = 500224
PBLK = 512
NBLK = PSPLIT // PBLK
NW = 32
BPW = BATCH // NW
CHUNK = 128
NCHUNK = BPW // CHUNK


def _relayout_body(u1, u2, i1, i2, pu, pi):
    pu[...] = jnp.concatenate([u1[...], u2[...]], axis=0).T
    pi[...] = jnp.concatenate([i1[...], i2[...]], axis=0).T


def _pair_tables(uT, iT):
    spec_lo = pl.BlockSpec((EDIM, PBLK), lambda i: (0, i))
    spec_hi = pl.BlockSpec((EDIM, PBLK), lambda i: (0, i + NBLK))
    out_spec = pl.BlockSpec((PBLK, 128), lambda i: (i, 0))
    return pl.pallas_call(
        _relayout_body,
        grid=(NBLK,),
        in_specs=[spec_lo, spec_hi, spec_lo, spec_hi],
        out_specs=[out_spec, out_spec],
        out_shape=[jax.ShapeDtypeStruct((PSPLIT, 128), jnp.float32)] * 2,
        compiler_params=pltpu.CompilerParams(
            dimension_semantics=("arbitrary",),
            fuse_transposed_lhs_in_matmul=True,
        ),
    )(uT, uT, iT, iT)


def _gather_body(ptab_u, ptab_i, uidx, iidx, out_u, out_i,
                 idx_v, rows_v, sem):
    wid = lax.axis_index("s") * 2 + lax.axis_index("c")
    base = wid * BPW

    def one_table(idx_hbm, ptab, out):
        pltpu.sync_copy(idx_hbm.at[wid], idx_v)
        for c in range(NCHUNK):
            for v in range(CHUNK // 16):
                sl = pl.ds(v * 16, 16)
                x = idx_v[c, sl]
                q = jnp.where(x >= PSPLIT, x - PSPLIT, x)
                idx_v[c, sl] = q
        cps = []
        for c in range(NCHUNK):
            cps.append(pltpu.async_copy(
                ptab.at[idx_v.at[c]],
                rows_v.at[pl.ds(c * CHUNK, CHUNK)], sem))
        for cp in cps:
            cp.wait()
        pltpu.sync_copy(rows_v, out.at[pl.ds(base, BPW)])

    one_table(uidx, ptab_u, out_u)
    one_table(iidx, ptab_i, out_i)


def _sc_gather(ptab_u, ptab_i, uidx, iidx):
    mesh = plsc.VectorSubcoreMesh(core_axis_name="c", subcore_axis_name="s")
    return pl.kernel(
        _gather_body,
        out_type=(
            jax.ShapeDtypeStruct((BATCH, 128), jnp.float32),
            jax.ShapeDtypeStruct((BATCH, 128), jnp.float32),
        ),
        mesh=mesh,
        scratch_types=[
            pltpu.VMEM((NCHUNK, CHUNK), jnp.int32),
            pltpu.VMEM((BPW, 128), jnp.float32),
            pltpu.SemaphoreType.DMA,
        ],
    )(ptab_u, ptab_i, uidx, iidx)


def _mlp_body(ur_ref, ir_ref, uh_ref, ih_ref, w1_ref, b1_ref, w2_ref, b2_ref,
              w3_ref, b3_ref, w4_ref, b4_ref, out_ref):
    dn = (((1,), (1,)), ((), ()))
    ur = ur_ref[...]
    ir = ir_ref[...]
    ue = jnp.where(uh_ref[...] >= PSPLIT, ur[:, EDIM:], ur[:, :EDIM])
    ie = jnp.where(ih_ref[...] >= PSPLIT, ir[:, EDIM:], ir[:, :EDIM])
    w1 = w1_ref[...]
    h = (lax.dot_general(ue, w1[:, :EDIM], dn,
                         preferred_element_type=jnp.float32)
         + lax.dot_general(ie, w1[:, EDIM:], dn,
                           preferred_element_type=jnp.float32)
         + b1_ref[...])
    h = jnp.maximum(h, 0.0)
    h = lax.dot_general(h, w2_ref[...], dn,
                        preferred_element_type=jnp.float32) + b2_ref[...]
    h = jnp.maximum(h, 0.0)
    h = lax.dot_general(h, w3_ref[...], dn,
                        preferred_element_type=jnp.float32) + b3_ref[...]
    h = jnp.maximum(h, 0.0)
    out_ref[...] = (jnp.sum(h * w4_ref[...], axis=1, keepdims=True)
                    + b4_ref[0, 0])


def _tc_mlp(ur, ir, uidx2, iidx2, W1, b1, W2, b2, W3, b3, W4, b4, block=2048):
    nblk = BATCH // block
    full = lambda shape: pl.BlockSpec(shape, lambda i: (0,) * len(shape))
    return pl.pallas_call(
        _mlp_body,
        grid=(nblk,),
        in_specs=[
            pl.BlockSpec((block, 128), lambda i: (i, 0)),
            pl.BlockSpec((block, 128), lambda i: (i, 0)),
            pl.BlockSpec((block, 1), lambda i: (i, 0)),
            pl.BlockSpec((block, 1), lambda i: (i, 0)),
            full(W1.shape), full(b1.shape),
            full(W2.shape), full(b2.shape),
            full(W3.shape), full(b3.shape),
            full(W4.shape), full(b4.shape),
        ],
        out_specs=pl.BlockSpec((block, 1), lambda i: (i, 0)),
        out_shape=jax.ShapeDtypeStruct((BATCH, 1), jnp.float32),
        compiler_params=pltpu.CompilerParams(
            dimension_semantics=("arbitrary",),
        ),
    )(ur, ir, uidx2, iidx2, W1, b1, W2, b2, W3, b3, W4, b4)


@jax.jit
def kernel(user_idx, item_idx, user_table, item_table,
           W1, b1, W2, b2, W3, b3, W4, b4):
    uT = user_table.T
    iT = item_table.T
    ptab_u, ptab_i = _pair_tables(uT, iT)
    uidx = user_idx.astype(jnp.int32)
    iidx = item_idx.astype(jnp.int32)
    ur, ir = _sc_gather(ptab_u, ptab_i,
                        uidx.reshape(NW, NCHUNK, CHUNK),
                        iidx.reshape(NW, NCHUNK, CHUNK))
    return _tc_mlp(ur, ir,
                   uidx.reshape(BATCH, 1), iidx.reshape(BATCH, 1),
                   W1, b1.reshape(1, -1), W2, b2.reshape(1, -1),
                   W3, b3.reshape(1, -1), W4, b4.reshape(1, -1))


# speedup vs baseline: 2.0226x; 1.4435x over previous
import jax
import jax.numpy as jnp
from jax import lax
from jax.experimental import pallas as pl
from jax.experimental.pallas import tpu as pltpu
from jax.experimental.pallas import tpu_sc as plsc

BATCH = 16384
EDIM = 64
NROWS = 1000000
PBLK = 512
NBLK = 489
QS = PBLK * NBLK
NW = 32
BPW = BATCH // NW
CHUNK = 128
NCHUNK = BPW // CHUNK

MASK_HI = -65536
ROUND_BIAS = 0x7FFF


def _bf16_hi(x):
    v = lax.bitcast_convert_type(x, jnp.int32)
    r = v + ROUND_BIAS + (lax.shift_right_logical(v, 16) & 1)
    return r & MASK_HI


def _relayout_body(ua, ub, uc, ud, ia, ib, ic, id_, pu, pi):
    def quad(a, b, c, d):
        t_ab = jnp.concatenate([a[...], b[...]], axis=0).T
        t_cd = jnp.concatenate([c[...], d[...]], axis=0).T
        hi = _bf16_hi(t_ab)
        lo = lax.shift_right_logical(_bf16_hi(t_cd), 16)
        return hi | lo

    pu[...] = quad(ua, ub, uc, ud)
    pi[...] = quad(ia, ib, ic, id_)


def _quad_tables(uT, iT):
    last = NROWS // PBLK - 1
    win = lambda k: pl.BlockSpec(
        (EDIM, PBLK), lambda i, k=k: (0, jnp.minimum(i + k * NBLK, last)))
    out_spec = pl.BlockSpec((PBLK, 128), lambda i: (i, 0))
    return pl.pallas_call(
        _relayout_body,
        grid=(NBLK,),
        in_specs=[win(0), win(1), win(2), win(3)] * 2,
        out_specs=[out_spec, out_spec],
        out_shape=[jax.ShapeDtypeStruct((QS, 128), jnp.int32)] * 2,
        compiler_params=pltpu.CompilerParams(
            dimension_semantics=("arbitrary",),
        ),
    )(uT, uT, uT, uT, iT, iT, iT, iT)


def _gather_body(ptab_u, ptab_i, uidx, iidx, out_u, out_i,
                 idx_v, rows_v, sem):
    wid = lax.axis_index("s") * 2 + lax.axis_index("c")
    base = wid * BPW

    def one_table(idx_hbm, ptab, out):
        pltpu.sync_copy(idx_hbm.at[wid], idx_v)
        for c in range(NCHUNK):
            for v in range(CHUNK // 16):
                sl = pl.ds(v * 16, 16)
                x = idx_v[c, sl]
                x = jnp.where(x >= 2 * QS, x - 2 * QS, x)
                x = jnp.where(x >= QS, x - QS, x)
                idx_v[c, sl] = x
        cps = []
        for c in range(NCHUNK):
            cps.append(pltpu.async_copy(
                ptab.at[idx_v.at[c]],
                rows_v.at[pl.ds(c * CHUNK, CHUNK)], sem))
        for cp in cps:
            cp.wait()
        pltpu.sync_copy(rows_v, out.at[pl.ds(base, BPW)])

    one_table(uidx, ptab_u, out_u)
    one_table(iidx, ptab_i, out_i)


def _sc_gather(ptab_u, ptab_i, uidx, iidx):
    mesh = plsc.VectorSubcoreMesh(core_axis_name="c", subcore_axis_name="s")
    return pl.kernel(
        _gather_body,
        out_type=(
            jax.ShapeDtypeStruct((BATCH, 128), jnp.int32),
            jax.ShapeDtypeStruct((BATCH, 128), jnp.int32),
        ),
        mesh=mesh,
        scratch_types=[
            pltpu.VMEM((NCHUNK, CHUNK), jnp.int32),
            pltpu.VMEM((BPW, 128), jnp.int32),
            pltpu.SemaphoreType.DMA,
        ],
    )(ptab_u, ptab_i, uidx, iidx)


def _decode(x, idx):
    ge2 = idx >= 2 * QS
    m = jnp.where(ge2, idx - 2 * QS, idx)
    odd = m >= QS
    xs = jnp.where(odd, x[:, EDIM:], x[:, :EDIM])
    w = jnp.where(ge2, lax.shift_left(xs, 16), xs & MASK_HI)
    return lax.bitcast_convert_type(w, jnp.float32)


def _mlp_body(ur_ref, ir_ref, uh_ref, ih_ref, w1_ref, b1_ref, w2_ref, b2_ref,
              w3_ref, b3_ref, w4_ref, b4_ref, out_ref):
    dn = (((1,), (1,)), ((), ()))
    ue = _decode(ur_ref[...], uh_ref[...])
    ie = _decode(ir_ref[...], ih_ref[...])
    w1 = w1_ref[...]
    h = (lax.dot_general(ue, w1[:, :EDIM], dn,
                         preferred_element_type=jnp.float32)
         + lax.dot_general(ie, w1[:, EDIM:], dn,
                           preferred_element_type=jnp.float32)
         + b1_ref[...])
    h = jnp.maximum(h, 0.0)
    h = lax.dot_general(h, w2_ref[...], dn,
                        preferred_element_type=jnp.float32) + b2_ref[...]
    h = jnp.maximum(h, 0.0)
    h = lax.dot_general(h, w3_ref[...], dn,
                        preferred_element_type=jnp.float32) + b3_ref[...]
    h = jnp.maximum(h, 0.0)
    out_ref[...] = (jnp.sum(h * w4_ref[...], axis=1, keepdims=True)
                    + b4_ref[0, 0])


def _tc_mlp(ur, ir, uidx2, iidx2, W1, b1, W2, b2, W3, b3, W4, b4, block=2048):
    nblk = BATCH // block
    full = lambda shape: pl.BlockSpec(shape, lambda i: (0,) * len(shape))
    return pl.pallas_call(
        _mlp_body,
        grid=(nblk,),
        in_specs=[
            pl.BlockSpec((block, 128), lambda i: (i, 0)),
            pl.BlockSpec((block, 128), lambda i: (i, 0)),
            pl.BlockSpec((block, 1), lambda i: (i, 0)),
            pl.BlockSpec((block, 1), lambda i: (i, 0)),
            full(W1.shape), full(b1.shape),
            full(W2.shape), full(b2.shape),
            full(W3.shape), full(b3.shape),
            full(W4.shape), full(b4.shape),
        ],
        out_specs=pl.BlockSpec((block, 1), lambda i: (i, 0)),
        out_shape=jax.ShapeDtypeStruct((BATCH, 1), jnp.float32),
        compiler_params=pltpu.CompilerParams(
            dimension_semantics=("arbitrary",),
        ),
    )(ur, ir, uidx2, iidx2, W1, b1, W2, b2, W3, b3, W4, b4)


@jax.jit
def kernel(user_idx, item_idx, user_table, item_table,
           W1, b1, W2, b2, W3, b3, W4, b4):
    uT = user_table.T
    iT = item_table.T
    ptab_u, ptab_i = _quad_tables(uT, iT)
    uidx = user_idx.astype(jnp.int32)
    iidx = item_idx.astype(jnp.int32)
    ur, ir = _sc_gather(ptab_u, ptab_i,
                        uidx.reshape(NW, NCHUNK, CHUNK),
                        iidx.reshape(NW, NCHUNK, CHUNK))
    return _tc_mlp(ur, ir,
                   uidx.reshape(BATCH, 1), iidx.reshape(BATCH, 1),
                   W1, b1.reshape(1, -1), W2, b2.reshape(1, -1),
                   W3, b3.reshape(1, -1), W4, b4.reshape(1, -1))


# speedup vs baseline: 2.8285x; 1.3985x over previous
import jax
import jax.numpy as jnp
from jax import lax
from jax.experimental import pallas as pl
from jax.experimental.pallas import tpu as pltpu
from jax.experimental.pallas import tpu_sc as plsc

BATCH = 16384
EDIM = 64
NROWS = 1000000
PBLK = 1024
NBLK = 245
QS = PBLK * NBLK
NW = 32
BPW = BATCH // NW
CHUNK = 128
NCHUNK = BPW // CHUNK

MASK_HI = -65536
ROUND_BIAS = 0x7FFF


def _bf16_hi(x):
    v = lax.bitcast_convert_type(x, jnp.int32)
    r = v + ROUND_BIAS + (lax.shift_right_logical(v, 16) & 1)
    return r & MASK_HI


def _relayout_body(ua, ub, uc, ud, ia, ib, ic, id_, pu, pi):
    def quad(a, b, c, d):
        t_ab = jnp.concatenate([a[...], b[...]], axis=0).T
        t_cd = jnp.concatenate([c[...], d[...]], axis=0).T
        hi = _bf16_hi(t_ab)
        lo = lax.shift_right_logical(_bf16_hi(t_cd), 16)
        return hi | lo

    pu[...] = quad(ua, ub, uc, ud)
    pi[...] = quad(ia, ib, ic, id_)


def _quad_tables(uT, iT):
    last = NROWS // PBLK - 1
    win = lambda k: pl.BlockSpec(
        (EDIM, PBLK), lambda i, k=k: (0, jnp.minimum(i + k * NBLK, last)))
    out_spec = pl.BlockSpec((PBLK, 128), lambda i: (i, 0))
    return pl.pallas_call(
        _relayout_body,
        grid=(NBLK,),
        in_specs=[win(0), win(1), win(2), win(3)] * 2,
        out_specs=[out_spec, out_spec],
        out_shape=[jax.ShapeDtypeStruct((QS, 128), jnp.int32)] * 2,
        compiler_params=pltpu.CompilerParams(
            dimension_semantics=("arbitrary",),
        ),
    )(uT, uT, uT, uT, iT, iT, iT, iT)


def _gather_body(ptab_u, ptab_i, uidx, iidx, out_u, out_i,
                 idx_v, rows_v, sem):
    wid = lax.axis_index("s") * 2 + lax.axis_index("c")
    base = wid * BPW

    def one_table(idx_hbm, ptab, out):
        pltpu.sync_copy(idx_hbm.at[wid], idx_v)
        for c in range(NCHUNK):
            for v in range(CHUNK // 16):
                sl = pl.ds(v * 16, 16)
                x = idx_v[c, sl]
                x = jnp.where(x >= 2 * QS, x - 2 * QS, x)
                x = jnp.where(x >= QS, x - QS, x)
                idx_v[c, sl] = x
        cps = []
        for c in range(NCHUNK):
            cps.append(pltpu.async_copy(
                ptab.at[idx_v.at[c]],
                rows_v.at[pl.ds(c * CHUNK, CHUNK)], sem))
        for cp in cps:
            cp.wait()
        pltpu.sync_copy(rows_v, out.at[pl.ds(base, BPW)])

    one_table(uidx, ptab_u, out_u)
    one_table(iidx, ptab_i, out_i)


def _sc_gather(ptab_u, ptab_i, uidx, iidx):
    mesh = plsc.VectorSubcoreMesh(core_axis_name="c", subcore_axis_name="s")
    return pl.kernel(
        _gather_body,
        out_type=(
            jax.ShapeDtypeStruct((BATCH, 128), jnp.int32),
            jax.ShapeDtypeStruct((BATCH, 128), jnp.int32),
        ),
        mesh=mesh,
        scratch_types=[
            pltpu.VMEM((NCHUNK, CHUNK), jnp.int32),
            pltpu.VMEM((BPW, 128), jnp.int32),
            pltpu.SemaphoreType.DMA,
        ],
    )(ptab_u, ptab_i, uidx, iidx)


def _decode(x, idx):
    ge2 = idx >= 2 * QS
    m = jnp.where(ge2, idx - 2 * QS, idx)
    odd = m >= QS
    xs = jnp.where(odd, x[:, EDIM:], x[:, :EDIM])
    w = jnp.where(ge2, lax.shift_left(xs, 16), xs & MASK_HI)
    return lax.bitcast_convert_type(w, jnp.float32)


def _mlp_body(ur_ref, ir_ref, uh_ref, ih_ref, w1_ref, b1_ref, w2_ref, b2_ref,
              w3_ref, b3_ref, w4_ref, b4_ref, out_ref):
    dn = (((1,), (1,)), ((), ()))
    ue = _decode(ur_ref[...], uh_ref[...])
    ie = _decode(ir_ref[...], ih_ref[...])
    w1 = w1_ref[...]
    h = (lax.dot_general(ue, w1[:, :EDIM], dn,
                         preferred_element_type=jnp.float32)
         + lax.dot_general(ie, w1[:, EDIM:], dn,
                           preferred_element_type=jnp.float32)
         + b1_ref[...])
    h = jnp.maximum(h, 0.0)
    h = lax.dot_general(h, w2_ref[...], dn,
                        preferred_element_type=jnp.float32) + b2_ref[...]
    h = jnp.maximum(h, 0.0)
    h = lax.dot_general(h, w3_ref[...], dn,
                        preferred_element_type=jnp.float32) + b3_ref[...]
    h = jnp.maximum(h, 0.0)
    out_ref[...] = (jnp.sum(h * w4_ref[...], axis=1, keepdims=True)
                    + b4_ref[0, 0])


def _tc_mlp(ur, ir, uidx2, iidx2, W1, b1, W2, b2, W3, b3, W4, b4, block=2048):
    nblk = BATCH // block
    full = lambda shape: pl.BlockSpec(shape, lambda i: (0,) * len(shape))
    return pl.pallas_call(
        _mlp_body,
        grid=(nblk,),
        in_specs=[
            pl.BlockSpec((block, 128), lambda i: (i, 0)),
            pl.BlockSpec((block, 128), lambda i: (i, 0)),
            pl.BlockSpec((block, 1), lambda i: (i, 0)),
            pl.BlockSpec((block, 1), lambda i: (i, 0)),
            full(W1.shape), full(b1.shape),
            full(W2.shape), full(b2.shape),
            full(W3.shape), full(b3.shape),
            full(W4.shape), full(b4.shape),
        ],
        out_specs=pl.BlockSpec((block, 1), lambda i: (i, 0)),
        out_shape=jax.ShapeDtypeStruct((BATCH, 1), jnp.float32),
        compiler_params=pltpu.CompilerParams(
            dimension_semantics=("arbitrary",),
        ),
    )(ur, ir, uidx2, iidx2, W1, b1, W2, b2, W3, b3, W4, b4)


@jax.jit
def kernel(user_idx, item_idx, user_table, item_table,
           W1, b1, W2, b2, W3, b3, W4, b4):
    uT = user_table.T
    iT = item_table.T
    ptab_u, ptab_i = _quad_tables(uT, iT)
    uidx = user_idx.astype(jnp.int32)
    iidx = item_idx.astype(jnp.int32)
    ur, ir = _sc_gather(ptab_u, ptab_i,
                        uidx.reshape(NW, NCHUNK, CHUNK),
                        iidx.reshape(NW, NCHUNK, CHUNK))
    return _tc_mlp(ur, ir,
                   uidx.reshape(BATCH, 1), iidx.reshape(BATCH, 1),
                   W1, b1.reshape(1, -1), W2, b2.reshape(1, -1),
                   W3, b3.reshape(1, -1), W4, b4.reshape(1, -1))


# speedup vs baseline: 3.4170x; 1.2080x over previous
import jax
import jax.numpy as jnp
from jax import lax
from jax.experimental import pallas as pl
from jax.experimental.pallas import tpu as pltpu
from jax.experimental.pallas import tpu_sc as plsc

BATCH = 16384
EDIM = 64
NROWS = 1000000
PBLK = 2048
NBLK = 123
QS = PBLK * NBLK
NW = 32
BPW = BATCH // NW
CHUNK = 128
NCHUNK = BPW // CHUNK

MASK_HI = -65536
ROUND_BIAS = 0x7FFF


def _bf16_hi(x):
    v = lax.bitcast_convert_type(x, jnp.int32)
    r = v + ROUND_BIAS + (lax.shift_right_logical(v, 16) & 1)
    return r & MASK_HI


def _relayout_body(ua, ub, uc, ud, ia, ib, ic, id_, pu, pi):
    def quad(a, b, c, d):
        t_ab = jnp.concatenate([a[...], b[...]], axis=0).T
        t_cd = jnp.concatenate([c[...], d[...]], axis=0).T
        hi = _bf16_hi(t_ab)
        lo = lax.shift_right_logical(_bf16_hi(t_cd), 16)
        return hi | lo

    pu[...] = quad(ua, ub, uc, ud)
    pi[...] = quad(ia, ib, ic, id_)


def _quad_tables(uT, iT):
    last = NROWS // PBLK - 1
    win = lambda k: pl.BlockSpec(
        (EDIM, PBLK), lambda i, k=k: (0, jnp.minimum(i + k * NBLK, last)))
    out_spec = pl.BlockSpec((PBLK, 128), lambda i: (i, 0))
    return pl.pallas_call(
        _relayout_body,
        grid=(NBLK,),
        in_specs=[win(0), win(1), win(2), win(3)] * 2,
        out_specs=[out_spec, out_spec],
        out_shape=[jax.ShapeDtypeStruct((QS, 128), jnp.int32)] * 2,
        compiler_params=pltpu.CompilerParams(
            dimension_semantics=("arbitrary",),
        ),
    )(uT, uT, uT, uT, iT, iT, iT, iT)


def _gather_body(ptab_u, ptab_i, uidx, iidx, out_u, out_i,
                 idx_v, rows_v, sem):
    wid = lax.axis_index("s") * 2 + lax.axis_index("c")
    base = wid * BPW

    def one_table(idx_hbm, ptab, out):
        pltpu.sync_copy(idx_hbm.at[wid], idx_v)
        for c in range(NCHUNK):
            for v in range(CHUNK // 16):
                sl = pl.ds(v * 16, 16)
                x = idx_v[c, sl]
                x = jnp.where(x >= 2 * QS, x - 2 * QS, x)
                x = jnp.where(x >= QS, x - QS, x)
                idx_v[c, sl] = x
        cps = []
        for c in range(NCHUNK):
            cps.append(pltpu.async_copy(
                ptab.at[idx_v.at[c]],
                rows_v.at[pl.ds(c * CHUNK, CHUNK)], sem))
        for cp in cps:
            cp.wait()
        pltpu.sync_copy(rows_v, out.at[pl.ds(base, BPW)])

    one_table(uidx, ptab_u, out_u)
    one_table(iidx, ptab_i, out_i)


def _sc_gather(ptab_u, ptab_i, uidx, iidx):
    mesh = plsc.VectorSubcoreMesh(core_axis_name="c", subcore_axis_name="s")
    return pl.kernel(
        _gather_body,
        out_type=(
            jax.ShapeDtypeStruct((BATCH, 128), jnp.int32),
            jax.ShapeDtypeStruct((BATCH, 128), jnp.int32),
        ),
        mesh=mesh,
        scratch_types=[
            pltpu.VMEM((NCHUNK, CHUNK), jnp.int32),
            pltpu.VMEM((BPW, 128), jnp.int32),
            pltpu.SemaphoreType.DMA,
        ],
    )(ptab_u, ptab_i, uidx, iidx)


def _decode(x, idx):
    ge2 = idx >= 2 * QS
    m = jnp.where(ge2, idx - 2 * QS, idx)
    odd = m >= QS
    xs = jnp.where(odd, x[:, EDIM:], x[:, :EDIM])
    w = jnp.where(ge2, lax.shift_left(xs, 16), xs & MASK_HI)
    return lax.bitcast_convert_type(w, jnp.float32)


def _mlp_body(ur_ref, ir_ref, uh_ref, ih_ref, w1_ref, b1_ref, w2_ref, b2_ref,
              w3_ref, b3_ref, w4_ref, b4_ref, out_ref):
    dn = (((1,), (1,)), ((), ()))
    ue = _decode(ur_ref[...], uh_ref[...])
    ie = _decode(ir_ref[...], ih_ref[...])
    w1 = w1_ref[...]
    h = (lax.dot_general(ue, w1[:, :EDIM], dn,
                         preferred_element_type=jnp.float32)
         + lax.dot_general(ie, w1[:, EDIM:], dn,
                           preferred_element_type=jnp.float32)
         + b1_ref[...])
    h = jnp.maximum(h, 0.0)
    h = lax.dot_general(h, w2_ref[...], dn,
                        preferred_element_type=jnp.float32) + b2_ref[...]
    h = jnp.maximum(h, 0.0)
    h = lax.dot_general(h, w3_ref[...], dn,
                        preferred_element_type=jnp.float32) + b3_ref[...]
    h = jnp.maximum(h, 0.0)
    out_ref[...] = (jnp.sum(h * w4_ref[...], axis=1, keepdims=True)
                    + b4_ref[0, 0])


def _tc_mlp(ur, ir, uidx2, iidx2, W1, b1, W2, b2, W3, b3, W4, b4, block=2048):
    nblk = BATCH // block
    full = lambda shape: pl.BlockSpec(shape, lambda i: (0,) * len(shape))
    return pl.pallas_call(
        _mlp_body,
        grid=(nblk,),
        in_specs=[
            pl.BlockSpec((block, 128), lambda i: (i, 0)),
            pl.BlockSpec((block, 128), lambda i: (i, 0)),
            pl.BlockSpec((block, 1), lambda i: (i, 0)),
            pl.BlockSpec((block, 1), lambda i: (i, 0)),
            full(W1.shape), full(b1.shape),
            full(W2.shape), full(b2.shape),
            full(W3.shape), full(b3.shape),
            full(W4.shape), full(b4.shape),
        ],
        out_specs=pl.BlockSpec((block, 1), lambda i: (i, 0)),
        out_shape=jax.ShapeDtypeStruct((BATCH, 1), jnp.float32),
        compiler_params=pltpu.CompilerParams(
            dimension_semantics=("arbitrary",),
        ),
    )(ur, ir, uidx2, iidx2, W1, b1, W2, b2, W3, b3, W4, b4)


@jax.jit
def kernel(user_idx, item_idx, user_table, item_table,
           W1, b1, W2, b2, W3, b3, W4, b4):
    uT = user_table.T
    iT = item_table.T
    ptab_u, ptab_i = _quad_tables(uT, iT)
    uidx = user_idx.astype(jnp.int32)
    iidx = item_idx.astype(jnp.int32)
    ur, ir = _sc_gather(ptab_u, ptab_i,
                        uidx.reshape(NW, NCHUNK, CHUNK),
                        iidx.reshape(NW, NCHUNK, CHUNK))
    return _tc_mlp(ur, ir,
                   uidx.reshape(BATCH, 1), iidx.reshape(BATCH, 1),
                   W1, b1.reshape(1, -1), W2, b2.reshape(1, -1),
                   W3, b3.reshape(1, -1), W4, b4.reshape(1, -1))


# speedup vs baseline: 3.6691x; 1.0738x over previous
import jax
import jax.numpy as jnp
from jax import lax
from jax.experimental import pallas as pl
from jax.experimental.pallas import tpu as pltpu
from jax.experimental.pallas import tpu_sc as plsc

BATCH = 16384
EDIM = 64
NROWS = 1000000
PBLK = 4096
NBLK = 62
QS = PBLK * NBLK
NW = 32
BPW = BATCH // NW
CHUNK = 128
NCHUNK = BPW // CHUNK

MASK_HI = -65536
ROUND_BIAS = 0x7FFF


def _bf16_hi(x):
    v = lax.bitcast_convert_type(x, jnp.int32)
    r = v + ROUND_BIAS + (lax.shift_right_logical(v, 16) & 1)
    return r & MASK_HI


def _relayout_body(ua, ub, uc, ud, ia, ib, ic, id_, pu, pi):
    def quad(a, b, c, d):
        t_ab = jnp.concatenate([a[...], b[...]], axis=0).T
        t_cd = jnp.concatenate([c[...], d[...]], axis=0).T
        hi = _bf16_hi(t_ab)
        lo = lax.shift_right_logical(_bf16_hi(t_cd), 16)
        return hi | lo

    pu[...] = quad(ua, ub, uc, ud)
    pi[...] = quad(ia, ib, ic, id_)


def _quad_tables(uT, iT):
    last = NROWS // PBLK - 1
    win = lambda k: pl.BlockSpec(
        (EDIM, PBLK), lambda i, k=k: (0, jnp.minimum(i + k * NBLK, last)))
    out_spec = pl.BlockSpec((PBLK, 128), lambda i: (i, 0))
    return pl.pallas_call(
        _relayout_body,
        grid=(NBLK,),
        in_specs=[win(0), win(1), win(2), win(3)] * 2,
        out_specs=[out_spec, out_spec],
        out_shape=[jax.ShapeDtypeStruct((QS, 128), jnp.int32)] * 2,
        compiler_params=pltpu.CompilerParams(
            dimension_semantics=("arbitrary",),
        ),
    )(uT, uT, uT, uT, iT, iT, iT, iT)


def _gather_body(ptab_u, ptab_i, uidx, iidx, out_u, out_i,
                 idx_v, rows_v, sem):
    wid = lax.axis_index("s") * 2 + lax.axis_index("c")
    base = wid * BPW

    def one_table(idx_hbm, ptab, out):
        pltpu.sync_copy(idx_hbm.at[wid], idx_v)
        for c in range(NCHUNK):
            for v in range(CHUNK // 16):
                sl = pl.ds(v * 16, 16)
                x = idx_v[c, sl]
                x = jnp.where(x >= 2 * QS, x - 2 * QS, x)
                x = jnp.where(x >= QS, x - QS, x)
                idx_v[c, sl] = x
        cps = []
        for c in range(NCHUNK):
            cps.append(pltpu.async_copy(
                ptab.at[idx_v.at[c]],
                rows_v.at[pl.ds(c * CHUNK, CHUNK)], sem))
        for cp in cps:
            cp.wait()
        pltpu.sync_copy(rows_v, out.at[pl.ds(base, BPW)])

    one_table(uidx, ptab_u, out_u)
    one_table(iidx, ptab_i, out_i)


def _sc_gather(ptab_u, ptab_i, uidx, iidx):
    mesh = plsc.VectorSubcoreMesh(core_axis_name="c", subcore_axis_name="s")
    return pl.kernel(
        _gather_body,
        out_type=(
            jax.ShapeDtypeStruct((BATCH, 128), jnp.int32),
            jax.ShapeDtypeStruct((BATCH, 128), jnp.int32),
        ),
        mesh=mesh,
        scratch_types=[
            pltpu.VMEM((NCHUNK, CHUNK), jnp.int32),
            pltpu.VMEM((BPW, 128), jnp.int32),
            pltpu.SemaphoreType.DMA,
        ],
    )(ptab_u, ptab_i, uidx, iidx)


def _decode(x, idx):
    ge2 = idx >= 2 * QS
    m = jnp.where(ge2, idx - 2 * QS, idx)
    odd = m >= QS
    xs = jnp.where(odd, x[:, EDIM:], x[:, :EDIM])
    w = jnp.where(ge2, lax.shift_left(xs, 16), xs & MASK_HI)
    return lax.bitcast_convert_type(w, jnp.float32)


def _mlp_body(ur_ref, ir_ref, uh_ref, ih_ref, w1_ref, b1_ref, w2_ref, b2_ref,
              w3_ref, b3_ref, w4_ref, b4_ref, out_ref):
    dn = (((1,), (1,)), ((), ()))
    ue = _decode(ur_ref[...], uh_ref[...])
    ie = _decode(ir_ref[...], ih_ref[...])
    w1 = w1_ref[...]
    h = (lax.dot_general(ue, w1[:, :EDIM], dn,
                         preferred_element_type=jnp.float32)
         + lax.dot_general(ie, w1[:, EDIM:], dn,
                           preferred_element_type=jnp.float32)
         + b1_ref[...])
    h = jnp.maximum(h, 0.0)
    h = lax.dot_general(h, w2_ref[...], dn,
                        preferred_element_type=jnp.float32) + b2_ref[...]
    h = jnp.maximum(h, 0.0)
    h = lax.dot_general(h, w3_ref[...], dn,
                        preferred_element_type=jnp.float32) + b3_ref[...]
    h = jnp.maximum(h, 0.0)
    out_ref[...] = (jnp.sum(h * w4_ref[...], axis=1, keepdims=True)
                    + b4_ref[0, 0])


def _tc_mlp(ur, ir, uidx2, iidx2, W1, b1, W2, b2, W3, b3, W4, b4, block=2048):
    nblk = BATCH // block
    full = lambda shape: pl.BlockSpec(shape, lambda i: (0,) * len(shape))
    return pl.pallas_call(
        _mlp_body,
        grid=(nblk,),
        in_specs=[
            pl.BlockSpec((block, 128), lambda i: (i, 0)),
            pl.BlockSpec((block, 128), lambda i: (i, 0)),
            pl.BlockSpec((block, 1), lambda i: (i, 0)),
            pl.BlockSpec((block, 1), lambda i: (i, 0)),
            full(W1.shape), full(b1.shape),
            full(W2.shape), full(b2.shape),
            full(W3.shape), full(b3.shape),
            full(W4.shape), full(b4.shape),
        ],
        out_specs=pl.BlockSpec((block, 1), lambda i: (i, 0)),
        out_shape=jax.ShapeDtypeStruct((BATCH, 1), jnp.float32),
        compiler_params=pltpu.CompilerParams(
            dimension_semantics=("arbitrary",),
        ),
    )(ur, ir, uidx2, iidx2, W1, b1, W2, b2, W3, b3, W4, b4)


@jax.jit
def kernel(user_idx, item_idx, user_table, item_table,
           W1, b1, W2, b2, W3, b3, W4, b4):
    uT = user_table.T
    iT = item_table.T
    ptab_u, ptab_i = _quad_tables(uT, iT)
    uidx = user_idx.astype(jnp.int32)
    iidx = item_idx.astype(jnp.int32)
    ur, ir = _sc_gather(ptab_u, ptab_i,
                        uidx.reshape(NW, NCHUNK, CHUNK),
                        iidx.reshape(NW, NCHUNK, CHUNK))
    return _tc_mlp(ur, ir,
                   uidx.reshape(BATCH, 1), iidx.reshape(BATCH, 1),
                   W1, b1.reshape(1, -1), W2, b2.reshape(1, -1),
                   W3, b3.reshape(1, -1), W4, b4.reshape(1, -1))


# speedup vs baseline: 3.7001x; 1.0085x over previous
import jax
import jax.numpy as jnp
from jax import lax
from jax.experimental import pallas as pl
from jax.experimental.pallas import tpu as pltpu
from jax.experimental.pallas import tpu_sc as plsc

BATCH = 16384
EDIM = 64
NROWS = 1000000
PBLK = 8192
NBLK = 31
QS = PBLK * NBLK
NW = 32
BPW = BATCH // NW
CHUNK = 128
NCHUNK = BPW // CHUNK

MASK_HI = -65536
ROUND_BIAS = 0x7FFF


def _bf16_hi(x):
    v = lax.bitcast_convert_type(x, jnp.int32)
    r = v + ROUND_BIAS + (lax.shift_right_logical(v, 16) & 1)
    return r & MASK_HI


def _relayout_body(ua, ub, uc, ud, ia, ib, ic, id_, pu, pi):
    def quad(a, b, c, d):
        t_ab = jnp.concatenate([a[...], b[...]], axis=0).T
        t_cd = jnp.concatenate([c[...], d[...]], axis=0).T
        hi = _bf16_hi(t_ab)
        lo = lax.shift_right_logical(_bf16_hi(t_cd), 16)
        return hi | lo

    pu[...] = quad(ua, ub, uc, ud)
    pi[...] = quad(ia, ib, ic, id_)


def _quad_tables(uT, iT):
    last = NROWS // PBLK - 1
    win = lambda k: pl.BlockSpec(
        (EDIM, PBLK), lambda i, k=k: (0, jnp.minimum(i + k * NBLK, last)))
    out_spec = pl.BlockSpec((PBLK, 128), lambda i: (i, 0))
    return pl.pallas_call(
        _relayout_body,
        grid=(NBLK,),
        in_specs=[win(0), win(1), win(2), win(3)] * 2,
        out_specs=[out_spec, out_spec],
        out_shape=[jax.ShapeDtypeStruct((QS, 128), jnp.int32)] * 2,
        compiler_params=pltpu.CompilerParams(
            dimension_semantics=("arbitrary",),
        ),
    )(uT, uT, uT, uT, iT, iT, iT, iT)


def _gather_body(ptab_u, ptab_i, uidx, iidx, out_u, out_i,
                 idx_v, rows_v, sem):
    wid = lax.axis_index("s") * 2 + lax.axis_index("c")
    base = wid * BPW

    def one_table(idx_hbm, ptab, out):
        pltpu.sync_copy(idx_hbm.at[wid], idx_v)
        for c in range(NCHUNK):
            for v in range(CHUNK // 16):
                sl = pl.ds(v * 16, 16)
                x = idx_v[c, sl]
                x = jnp.where(x >= 2 * QS, x - 2 * QS, x)
                x = jnp.where(x >= QS, x - QS, x)
                idx_v[c, sl] = x
        cps = []
        for c in range(NCHUNK):
            cps.append(pltpu.async_copy(
                ptab.at[idx_v.at[c]],
                rows_v.at[pl.ds(c * CHUNK, CHUNK)], sem))
        for cp in cps:
            cp.wait()
        pltpu.sync_copy(rows_v, out.at[pl.ds(base, BPW)])

    one_table(uidx, ptab_u, out_u)
    one_table(iidx, ptab_i, out_i)


def _sc_gather(ptab_u, ptab_i, uidx, iidx):
    mesh = plsc.VectorSubcoreMesh(core_axis_name="c", subcore_axis_name="s")
    return pl.kernel(
        _gather_body,
        out_type=(
            jax.ShapeDtypeStruct((BATCH, 128), jnp.int32),
            jax.ShapeDtypeStruct((BATCH, 128), jnp.int32),
        ),
        mesh=mesh,
        scratch_types=[
            pltpu.VMEM((NCHUNK, CHUNK), jnp.int32),
            pltpu.VMEM((BPW, 128), jnp.int32),
            pltpu.SemaphoreType.DMA,
        ],
    )(ptab_u, ptab_i, uidx, iidx)


def _decode(x, idx):
    ge2 = idx >= 2 * QS
    m = jnp.where(ge2, idx - 2 * QS, idx)
    odd = m >= QS
    xs = jnp.where(odd, x[:, EDIM:], x[:, :EDIM])
    w = jnp.where(ge2, lax.shift_left(xs, 16), xs & MASK_HI)
    return lax.bitcast_convert_type(w, jnp.float32)


def _mlp_body(ur_ref, ir_ref, uh_ref, ih_ref, w1_ref, b1_ref, w2_ref, b2_ref,
              w3_ref, b3_ref, w4_ref, b4_ref, out_ref):
    dn = (((1,), (1,)), ((), ()))
    ue = _decode(ur_ref[...], uh_ref[...])
    ie = _decode(ir_ref[...], ih_ref[...])
    w1 = w1_ref[...]
    h = (lax.dot_general(ue, w1[:, :EDIM], dn,
                         preferred_element_type=jnp.float32)
         + lax.dot_general(ie, w1[:, EDIM:], dn,
                           preferred_element_type=jnp.float32)
         + b1_ref[...])
    h = jnp.maximum(h, 0.0)
    h = lax.dot_general(h, w2_ref[...], dn,
                        preferred_element_type=jnp.float32) + b2_ref[...]
    h = jnp.maximum(h, 0.0)
    h = lax.dot_general(h, w3_ref[...], dn,
                        preferred_element_type=jnp.float32) + b3_ref[...]
    h = jnp.maximum(h, 0.0)
    out_ref[...] = (jnp.sum(h * w4_ref[...], axis=1, keepdims=True)
                    + b4_ref[0, 0])


def _tc_mlp(ur, ir, uidx2, iidx2, W1, b1, W2, b2, W3, b3, W4, b4, block=2048):
    nblk = BATCH // block
    full = lambda shape: pl.BlockSpec(shape, lambda i: (0,) * len(shape))
    return pl.pallas_call(
        _mlp_body,
        grid=(nblk,),
        in_specs=[
            pl.BlockSpec((block, 128), lambda i: (i, 0)),
            pl.BlockSpec((block, 128), lambda i: (i, 0)),
            pl.BlockSpec((block, 1), lambda i: (i, 0)),
            pl.BlockSpec((block, 1), lambda i: (i, 0)),
            full(W1.shape), full(b1.shape),
            full(W2.shape), full(b2.shape),
            full(W3.shape), full(b3.shape),
            full(W4.shape), full(b4.shape),
        ],
        out_specs=pl.BlockSpec((block, 1), lambda i: (i, 0)),
        out_shape=jax.ShapeDtypeStruct((BATCH, 1), jnp.float32),
        compiler_params=pltpu.CompilerParams(
            dimension_semantics=("arbitrary",),
        ),
    )(ur, ir, uidx2, iidx2, W1, b1, W2, b2, W3, b3, W4, b4)


@jax.jit
def kernel(user_idx, item_idx, user_table, item_table,
           W1, b1, W2, b2, W3, b3, W4, b4):
    uT = user_table.T
    iT = item_table.T
    ptab_u, ptab_i = _quad_tables(uT, iT)
    uidx = user_idx.astype(jnp.int32)
    iidx = item_idx.astype(jnp.int32)
    ur, ir = _sc_gather(ptab_u, ptab_i,
                        uidx.reshape(NW, NCHUNK, CHUNK),
                        iidx.reshape(NW, NCHUNK, CHUNK))
    return _tc_mlp(ur, ir,
                   uidx.reshape(BATCH, 1), iidx.reshape(BATCH, 1),
                   W1, b1.reshape(1, -1), W2, b2.reshape(1, -1),
                   W3, b3.reshape(1, -1), W4, b4.reshape(1, -1))


# speedup vs baseline: 3.7397x; 1.0107x over previous
import jax
import jax.numpy as jnp
from jax import lax
from jax.experimental import pallas as pl
from jax.experimental.pallas import tpu as pltpu
from jax.experimental.pallas import tpu_sc as plsc

BATCH = 16384
EDIM = 64
NROWS = 1000000
PBLK = 8192
NBLK = 31
QS = PBLK * NBLK
NW = 32
BPW = BATCH // NW
CHUNK = 128
NCHUNK = BPW // CHUNK

MASK_HI = -65536
ROUND_BIAS = 0x7FFF


def _bf16_hi(x):
    v = lax.bitcast_convert_type(x, jnp.int32)
    r = v + ROUND_BIAS + (lax.shift_right_logical(v, 16) & 1)
    return r & MASK_HI


def _relayout_body(ua, ub, uc, ud, ia, ib, ic, id_, pu, pi):
    def quad(a, b, c, d):
        t_ab = jnp.concatenate([a[...], b[...]], axis=0).T
        t_cd = jnp.concatenate([c[...], d[...]], axis=0).T
        hi = _bf16_hi(t_ab)
        lo = lax.shift_right_logical(_bf16_hi(t_cd), 16)
        return hi | lo

    pu[...] = quad(ua, ub, uc, ud)
    pi[...] = quad(ia, ib, ic, id_)


def _quad_tables(uT, iT):
    last = NROWS // PBLK - 1
    win = lambda k: pl.BlockSpec(
        (EDIM, PBLK), lambda i, k=k: (0, jnp.minimum(i + k * NBLK, last)))
    out_spec = pl.BlockSpec((PBLK, 128), lambda i: (i, 0))
    return pl.pallas_call(
        _relayout_body,
        grid=(NBLK,),
        in_specs=[win(0), win(1), win(2), win(3)] * 2,
        out_specs=[out_spec, out_spec],
        out_shape=[jax.ShapeDtypeStruct((QS, 128), jnp.int32)] * 2,
        compiler_params=pltpu.CompilerParams(
            dimension_semantics=("arbitrary",),
        ),
    )(uT, uT, uT, uT, iT, iT, iT, iT)


def _gather_body(ptab_u, ptab_i, uidx, iidx, out_u, out_i,
                 idx_v, rows_v, sem):
    wid = lax.axis_index("s") * 2 + lax.axis_index("c")
    base = wid * BPW

    def one_table(idx_hbm, ptab, out):
        pltpu.sync_copy(idx_hbm.at[wid], idx_v)
        for c in range(NCHUNK):
            for v in range(CHUNK // 16):
                sl = pl.ds(v * 16, 16)
                x = idx_v[c, sl]
                x = jnp.where(x >= 2 * QS, x - 2 * QS, x)
                x = jnp.where(x >= QS, x - QS, x)
                idx_v[c, sl] = x
        cps = []
        for c in range(NCHUNK):
            cps.append(pltpu.async_copy(
                ptab.at[idx_v.at[c]],
                rows_v.at[pl.ds(c * CHUNK, CHUNK)], sem))
        for cp in cps:
            cp.wait()
        pltpu.sync_copy(rows_v, out.at[pl.ds(base, BPW)])

    one_table(uidx, ptab_u, out_u)
    one_table(iidx, ptab_i, out_i)


def _sc_gather(ptab_u, ptab_i, uidx, iidx):
    mesh = plsc.VectorSubcoreMesh(core_axis_name="c", subcore_axis_name="s")
    return pl.kernel(
        _gather_body,
        out_type=(
            jax.ShapeDtypeStruct((BATCH, 128), jnp.int32),
            jax.ShapeDtypeStruct((BATCH, 128), jnp.int32),
        ),
        mesh=mesh,
        scratch_types=[
            pltpu.VMEM((NCHUNK, CHUNK), jnp.int32),
            pltpu.VMEM((BPW, 128), jnp.int32),
            pltpu.SemaphoreType.DMA,
        ],
    )(ptab_u, ptab_i, uidx, iidx)


def _decode(x, idx):
    ge2 = idx >= 2 * QS
    m = jnp.where(ge2, idx - 2 * QS, idx)
    odd = m >= QS
    xs = jnp.where(odd, x[:, EDIM:], x[:, :EDIM])
    w = jnp.where(ge2, lax.shift_left(xs, 16), xs & MASK_HI)
    return lax.bitcast_convert_type(w, jnp.float32)


def _mlp_body(ur_ref, ir_ref, uh_ref, ih_ref, w1_ref, b1_ref, w2_ref, b2_ref,
              w3_ref, b3_ref, w4_ref, b4_ref, out_ref):
    dn = (((1,), (1,)), ((), ()))
    ue = _decode(ur_ref[...], uh_ref[...])
    ie = _decode(ir_ref[...], ih_ref[...])
    w1 = w1_ref[...]
    h = (lax.dot_general(ue, w1[:, :EDIM], dn,
                         preferred_element_type=jnp.float32)
         + lax.dot_general(ie, w1[:, EDIM:], dn,
                           preferred_element_type=jnp.float32)
         + b1_ref[...])
    h = jnp.maximum(h, 0.0)
    h = lax.dot_general(h, w2_ref[...], dn,
                        preferred_element_type=jnp.float32) + b2_ref[...]
    h = jnp.maximum(h, 0.0)
    h = lax.dot_general(h, w3_ref[...], dn,
                        preferred_element_type=jnp.float32) + b3_ref[...]
    h = jnp.maximum(h, 0.0)
    out_ref[...] = (jnp.sum(h * w4_ref[...], axis=1, keepdims=True)
                    + b4_ref[0, 0])


def _tc_mlp(ur, ir, uidx2, iidx2, W1, b1, W2, b2, W3, b3, W4, b4, block=4096):
    nblk = BATCH // block
    full = lambda shape: pl.BlockSpec(shape, lambda i: (0,) * len(shape))
    return pl.pallas_call(
        _mlp_body,
        grid=(nblk,),
        in_specs=[
            pl.BlockSpec((block, 128), lambda i: (i, 0)),
            pl.BlockSpec((block, 128), lambda i: (i, 0)),
            pl.BlockSpec((block, 1), lambda i: (i, 0)),
            pl.BlockSpec((block, 1), lambda i: (i, 0)),
            full(W1.shape), full(b1.shape),
            full(W2.shape), full(b2.shape),
            full(W3.shape), full(b3.shape),
            full(W4.shape), full(b4.shape),
        ],
        out_specs=pl.BlockSpec((block, 1), lambda i: (i, 0)),
        out_shape=jax.ShapeDtypeStruct((BATCH, 1), jnp.float32),
        compiler_params=pltpu.CompilerParams(
            dimension_semantics=("arbitrary",),
        ),
    )(ur, ir, uidx2, iidx2, W1, b1, W2, b2, W3, b3, W4, b4)


@jax.jit
def kernel(user_idx, item_idx, user_table, item_table,
           W1, b1, W2, b2, W3, b3, W4, b4):
    uT = user_table.T
    iT = item_table.T
    ptab_u, ptab_i = _quad_tables(uT, iT)
    uidx = user_idx.astype(jnp.int32)
    iidx = item_idx.astype(jnp.int32)
    ur, ir = _sc_gather(ptab_u, ptab_i,
                        uidx.reshape(NW, NCHUNK, CHUNK),
                        iidx.reshape(NW, NCHUNK, CHUNK))
    return _tc_mlp(ur, ir,
                   uidx.reshape(BATCH, 1), iidx.reshape(BATCH, 1),
                   W1, b1.reshape(1, -1), W2, b2.reshape(1, -1),
                   W3, b3.reshape(1, -1), W4, b4.reshape(1, -1))
